# Initial kernel scaffold; baseline (speedup 1.0000x reference)
import jax
import jax.numpy as jnp
from jax.experimental import pallas as pl


def kernel(z, codebook):
    raise NotImplementedError("write your pallas kernel here")


# speedup vs baseline: 1.7374x; 1.7374x over previous
import jax
import jax.numpy as jnp
from jax.experimental import pallas as pl
from jax.experimental.pallas import tpu as pltpu

_K = 1024
_D = 64
_R = 2048
_CC = 0.25


def _vq_body(x_ref, cbt_ref, cb_ref, x2_ref, e2_ref, out_ref, loss_ref):
    i = pl.program_id(0)
    x = x_ref[...]
    cbt = cbt_ref[...]
    xe = jax.lax.dot_general(x, cbt, (((1,), (0,)), ((), ())),
                             preferred_element_type=jnp.float32)
    d = (x2_ref[...] + e2_ref[...]) - 2.0 * xe
    m = jnp.min(d, axis=1, keepdims=True)
    iota = jax.lax.broadcasted_iota(jnp.int32, (_R, _K), 1)
    idx = jnp.min(jnp.where(d == m, iota, _K), axis=1)
    onehot = (iota == idx[:, None]).astype(jnp.float32)
    q = jax.lax.dot_general(onehot, cb_ref[...], (((1,), (0,)), ((), ())),
                            preferred_element_type=jnp.float32)
    out_ref[...] = x + (q - x)
    part = jnp.sum((q - x) ** 2).reshape(1, 1)

    @pl.when(i == 0)
    def _init():
        loss_ref[...] = part

    @pl.when(i != 0)
    def _acc():
        loss_ref[...] += part


def kernel(z, codebook):
    n = z.shape[0] * z.shape[1]
    flat = z.reshape(n, _D)
    cbt = codebook.T
    x2 = jnp.sum(flat ** 2, axis=1, keepdims=True)
    e2 = jnp.sum(codebook ** 2, axis=1)[None, :]
    grid = n // _R
    out, loss_sum = pl.pallas_call(
        _vq_body,
        grid=(grid,),
        in_specs=[
            pl.BlockSpec((_R, _D), lambda i: (i, 0)),
            pl.BlockSpec((_D, _K), lambda i: (0, 0)),
            pl.BlockSpec((_K, _D), lambda i: (0, 0)),
            pl.BlockSpec((_R, 1), lambda i: (i, 0)),
            pl.BlockSpec((1, _K), lambda i: (0, 0)),
        ],
        out_specs=[
            pl.BlockSpec((_R, _D), lambda i: (i, 0)),
            pl.BlockSpec((1, 1), lambda i: (0, 0)),
        ],
        out_shape=[
            jax.ShapeDtypeStruct((n, _D), jnp.float32),
            jax.ShapeDtypeStruct((1, 1), jnp.float32),
        ],
    )(flat, cbt, codebook, x2, e2)
    m = loss_sum[0, 0] / (n * _D)
    loss = m + _CC * m
    return out.reshape(z.shape), loss


# speedup vs baseline: 1.9393x; 1.1162x over previous
import jax
import jax.numpy as jnp
from jax.experimental import pallas as pl
from jax.experimental.pallas import tpu as pltpu

_K = 1024
_D = 64
_R = 2048
_CC = 0.25


def _vq_body(x_ref, cbt2_ref, cb_ref, x2_ref, e2_ref, iota_ref, out_ref,
             loss_ref):
    i = pl.program_id(0)
    x = x_ref[...]
    xe2 = jax.lax.dot_general(x, cbt2_ref[...], (((1,), (0,)), ((), ())),
                              preferred_element_type=jnp.float32)
    d = (x2_ref[...] + e2_ref[...]) - xe2
    m = jnp.min(d, axis=1, keepdims=True)
    iota = iota_ref[...]
    idx = jnp.min(jnp.where(d == m, iota, float(_K)), axis=1)
    onehot = (iota == idx[:, None]).astype(jnp.float32)
    q = jax.lax.dot_general(onehot, cb_ref[...], (((1,), (0,)), ((), ())),
                            preferred_element_type=jnp.float32)
    out_ref[...] = x + (q - x)
    part = jnp.sum((q - x) ** 2).reshape(1, 1)

    @pl.when(i == 0)
    def _init():
        loss_ref[...] = part

    @pl.when(i != 0)
    def _acc():
        loss_ref[...] += part


def kernel(z, codebook):
    n = z.shape[0] * z.shape[1]
    flat = z.reshape(n, _D)
    cbt2 = 2.0 * codebook.T
    x2 = jnp.sum(flat ** 2, axis=1, keepdims=True)
    e2 = jnp.sum(codebook ** 2, axis=1)[None, :]
    grid = n // _R
    out, loss_sum = pl.pallas_call(
        _vq_body,
        grid=(grid,),
        in_specs=[
            pl.BlockSpec((_R, _D), lambda i: (i, 0)),
            pl.BlockSpec((_D, _K), lambda i: (0, 0)),
            pl.BlockSpec((_K, _D), lambda i: (0, 0)),
            pl.BlockSpec((_R, 1), lambda i: (i, 0)),
            pl.BlockSpec((1, _K), lambda i: (0, 0)),
            pl.BlockSpec((1, _K), lambda i: (0, 0)),
        ],
        out_specs=[
            pl.BlockSpec((_R, _D), lambda i: (i, 0)),
            pl.BlockSpec((1, 1), lambda i: (0, 0)),
        ],
        out_shape=[
            jax.ShapeDtypeStruct((n, _D), jnp.float32),
            jax.ShapeDtypeStruct((1, 1), jnp.float32),
        ],
    )(flat, cbt2, codebook, x2, e2,
      jnp.arange(_K, dtype=jnp.float32)[None, :])
    m = loss_sum[0, 0] / (n * _D)
    loss = m + _CC * m
    return out.reshape(z.shape), loss
